# Initial kernel scaffold; baseline (speedup 1.0000x reference)
import jax
import jax.numpy as jnp
from jax.experimental import pallas as pl


def kernel(game_x, state_x, pc_x, edge_index_v_v, edge_type_v_v, edge_index_history_v_s, edge_index_history_s_v, edge_attr_history_v_s, edge_index_in_v_s, edge_index_in_s_v, edge_index_s_s, edge_index_pc_pc, edge_index_pc_s, edge_index_s_pc, params):
    raise NotImplementedError("write your pallas kernel here")


# speedup vs baseline: 2.8891x; 2.8891x over previous
import functools

import jax
import jax.numpy as jnp
from jax import lax
from jax.experimental import pallas as pl
from jax.experimental.pallas import tpu as pltpu
from jax.experimental.pallas import tpu_sc as plsc

N = 10000
H = 256
E = 160000
HF = 128
K = 128
NT = 16
EPAD = 163840
NCHUNK = EPAD // (NT * K)
ACC_N = 10240
TRASH = 10016
NLISTS = 9
BN_EPS = 1e-5

_mesh = plsc.VectorSubcoreMesh(core_axis_name="c", subcore_axis_name="s")


def _segsum_body(x2, src2, dst, zrows_hbm, out, idx_s, idx_d, rows, acc, sem):
    c = lax.axis_index("c")
    s = lax.axis_index("s")
    zr = ACC_N // NT
    pltpu.sync_copy(zrows_hbm.at[pl.ds(s * zr, zr)], acc.at[pl.ds(s * zr, zr)])
    plsc.subcore_barrier()
    ebase = s * (EPAD // NT)

    def chunk(g, carry):
        eb = pl.multiple_of(ebase + g * K, K)
        pltpu.sync_copy(src2.at[pl.ds(c * EPAD + eb, K)], idx_s)
        pltpu.sync_copy(dst.at[pl.ds(eb, K)], idx_d)
        pltpu.async_copy(x2.at[idx_s], rows, sem).wait()
        pltpu.sync_copy(rows, acc.at[idx_d], add=True)
        return carry

    lax.fori_loop(0, NCHUNK, chunk, 0)
    plsc.subcore_barrier()
    pltpu.sync_copy(acc.at[pl.ds(s * zr, zr)],
                    out.at[pl.ds(c * ACC_N + s * zr, zr), :])


_segsum = functools.partial(
    pl.kernel,
    out_type=jax.ShapeDtypeStruct((2 * ACC_N, HF), jnp.float32),
    mesh=_mesh,
    scratch_types=[
        pltpu.VMEM((K,), jnp.int32),
        pltpu.VMEM((K,), jnp.int32),
        pltpu.VMEM((K, HF), jnp.float32),
        pltpu.VMEM_SHARED((ACC_N, HF), jnp.float32),
        pltpu.SemaphoreType.DMA,
    ],
)(_segsum_body)


def _hist_body(dsts, ones_hbm, z16_hbm, out, idx_d, ones_v, h2d):
    c = lax.axis_index("c")
    s = lax.axis_index("s")
    pltpu.sync_copy(ones_hbm, ones_v)
    zr = ACC_N // NT
    for l in range(NLISTS):
        @pl.when(c == (l % 2))
        def _():
            pltpu.sync_copy(z16_hbm.at[pl.ds(s * zr, zr)],
                            h2d.at[pl.ds(s * zr, zr)])
            plsc.subcore_barrier()
            ebase = s * (EPAD // NT)

            def chunk(g, carry):
                eb = pl.multiple_of(ebase + g * K, K)
                pltpu.sync_copy(dsts.at[pl.ds(l * EPAD + eb, K)], idx_d)
                pltpu.sync_copy(ones_v, h2d.at[idx_d], add=True)
                return carry

            lax.fori_loop(0, NCHUNK, chunk, 0)
            plsc.subcore_barrier()
            pltpu.sync_copy(h2d.at[pl.ds(s * zr, zr)],
                            out.at[pl.ds(l * ACC_N + s * zr, zr), :])


_hist = functools.partial(
    pl.kernel,
    out_type=jax.ShapeDtypeStruct((NLISTS * ACC_N, 16), jnp.float32),
    mesh=_mesh,
    scratch_types=[
        pltpu.VMEM((K,), jnp.int32),
        pltpu.VMEM((K, 16), jnp.float32),
        pltpu.VMEM_SHARED((ACC_N, 16), jnp.float32),
    ],
)(_hist_body)


_BN_ROWS = 2000


def _row_spec(width):
    return pl.BlockSpec((_BN_ROWS, width), lambda i: (i, 0))


def _const_spec(shape):
    return pl.BlockSpec(shape, lambda i: tuple(0 for _ in shape))


def _tc_call(body, in_specs, out_width, grid=N // _BN_ROWS, seq=False):
    return pl.pallas_call(
        body,
        grid=(grid,),
        in_specs=in_specs,
        out_specs=_row_spec(out_width),
        out_shape=jax.ShapeDtypeStruct((N, out_width), jnp.float32),
        compiler_params=pltpu.CompilerParams(
            dimension_semantics=("arbitrary" if seq else "parallel",)),
    )


def _mm_body(x_ref, w_ref, sc_ref, o_ref):
    x = x_ref[...] * sc_ref[...]
    o_ref[...] = jnp.dot(x, w_ref[...], preferred_element_type=jnp.float32)


def _make_mm(kw):
    return _tc_call(_mm_body,
                    [_row_spec(kw), _const_spec((kw, H)), _row_spec(1)], H)


_mm128 = _make_mm(128)
_mm256 = _make_mm(256)


def _sage_body(yc_ref, ci_ref, xd_ref, wl_ref, bl_ref, wr_ref, o_ref):
    ci = ci_ref[...]
    t = (jnp.dot(yc_ref[0] * ci, wl_ref[0], preferred_element_type=jnp.float32)
         + jnp.dot(yc_ref[1] * ci, wl_ref[1], preferred_element_type=jnp.float32)
         + jnp.dot(xd_ref[...], wr_ref[...], preferred_element_type=jnp.float32)
         + bl_ref[...])
    nrm = jnp.maximum(jnp.sqrt(jnp.sum(t * t, axis=-1, keepdims=True)), 1e-12)
    o_ref[...] = jnp.maximum(t / nrm, 0.0)


def _make_sage(dw):
    return _tc_call(
        _sage_body,
        [pl.BlockSpec((2, _BN_ROWS, HF), lambda i: (0, i, 0)), _row_spec(1),
         _row_spec(dw), _const_spec((2, HF, H)), _const_spec((1, H)),
         _const_spec((dw, H))], H)


_sage128 = _make_sage(128)
_sage256 = _make_sage(256)


def _gcn_body(relu, yc_ref, hp_ref, dv_ref, b_ref, o_ref):
    y = jnp.concatenate([yc_ref[0], yc_ref[1]], axis=-1) + hp_ref[...]
    t = y * dv_ref[...] + b_ref[...]
    o_ref[...] = jnp.maximum(t, 0.0) if relu else t


def _make_gcn(relu):
    return _tc_call(
        functools.partial(_gcn_body, relu),
        [pl.BlockSpec((2, _BN_ROWS, HF), lambda i: (0, i, 0)), _row_spec(H),
         _row_spec(1), _const_spec((1, H))], H)


_gcn_relu = _make_gcn(True)
_gcn_lin = _make_gcn(False)


def _bnstat_body(x_ref, o_ref):
    @pl.when(pl.program_id(0) == 0)
    def _():
        o_ref[...] = jnp.zeros_like(o_ref)
    x = x_ref[...]
    o_ref[0, :] += jnp.sum(x, axis=0)
    o_ref[1, :] += jnp.sum(x * x, axis=0)


_bnstat = pl.pallas_call(
    _bnstat_body,
    grid=(N // _BN_ROWS,),
    in_specs=[_row_spec(H)],
    out_specs=_const_spec((2, H)),
    out_shape=jax.ShapeDtypeStruct((2, H), jnp.float32),
    compiler_params=pltpu.CompilerParams(dimension_semantics=("arbitrary",)),
)


def _bnapply_body(x_ref, st_ref, g_ref, b_ref, o_ref):
    m = st_ref[0, :] / N
    v = st_ref[1, :] / N - m * m
    scale = lax.rsqrt(v + BN_EPS) * g_ref[0, :]
    o_ref[...] = (x_ref[...] - m[None, :]) * scale[None, :] + b_ref[...]


_bnapply = _tc_call(
    _bnapply_body,
    [_row_spec(H), _const_spec((2, H)), _const_spec((1, H)),
     _const_spec((1, H))], H)


def _scale_body(cnt_ref, dinv_ref, cinv_ref):
    cnt = cnt_ref[...]
    dinv_ref[...] = lax.rsqrt(cnt + 1.0)
    cinv_ref[...] = 1.0 / jnp.maximum(cnt, 1.0)


_scale = pl.pallas_call(
    _scale_body,
    in_specs=[pl.BlockSpec((NLISTS, ACC_N), lambda: (0, 0))],
    out_specs=[pl.BlockSpec((NLISTS, ACC_N), lambda: (0, 0))] * 2,
    out_shape=[jax.ShapeDtypeStruct((NLISTS, ACC_N), jnp.float32)] * 2,
)


def _logits_body(x_ref, w1_ref, b1_ref, w2_ref, b2_ref, o_ref):
    t = jnp.dot(x_ref[...], w1_ref[...], preferred_element_type=jnp.float32)
    t = jnp.maximum(t + b1_ref[...], 0.0)
    o_ref[...] = jnp.dot(t, w2_ref[...],
                         preferred_element_type=jnp.float32) + b2_ref[...]


_logits = _tc_call(
    _logits_body,
    [_row_spec(H), _const_spec((H, 128)), _const_spec((1, 128)),
     _const_spec((128, 128)), _const_spec((1, 128))], 128)


def _lsm_body(x_ref, o_ref):
    x = x_ref[:, 0:1]
    m = jnp.max(x)
    lse = m + jnp.log(jnp.sum(jnp.exp(x - m)))
    o_ref[...] = x - lse


_lsm = pl.pallas_call(
    _lsm_body,
    in_specs=[pl.BlockSpec((N, 128), lambda: (0, 0))],
    out_specs=pl.BlockSpec((N, 1), lambda: (0, 0)),
    out_shape=jax.ShapeDtypeStruct((N, 1), jnp.float32),
)


def _pad_cols(x, w):
    return jnp.pad(x, ((0, 0), (0, w - x.shape[1])))


def _prep_edges(ei):
    src = ei[0].astype(jnp.int32)
    dst = ei[1].astype(jnp.int32)
    srcp = jnp.concatenate([src, jnp.zeros((EPAD - E,), jnp.int32)])
    dstp = jnp.concatenate([dst, jnp.full((EPAD - E,), TRASH, jnp.int32)])
    src2 = jnp.concatenate([srcp * 2, srcp * 2 + 1])
    return src2, dstp


def kernel(game_x, state_x, pc_x, edge_index_v_v, edge_type_v_v,
           edge_index_history_v_s, edge_index_history_s_v,
           edge_attr_history_v_s, edge_index_in_v_s, edge_index_in_s_v,
           edge_index_s_s, edge_index_pc_pc, edge_index_pc_s,
           edge_index_s_pc, params):
    p = params
    del edge_type_v_v, edge_attr_history_v_s

    lists = {
        'ss': _prep_edges(edge_index_s_s),
        'vv': _prep_edges(edge_index_v_v),
        'pp': _prep_edges(edge_index_pc_pc),
        'hsv': _prep_edges(edge_index_history_s_v),
        'isv': _prep_edges(edge_index_in_s_v),
        'sp': _prep_edges(edge_index_s_pc),
        'hvs': _prep_edges(edge_index_history_v_s),
        'ivs': _prep_edges(edge_index_in_v_s),
        'ps': _prep_edges(edge_index_pc_s),
    }
    order = ['ss', 'vv', 'pp', 'hsv', 'isv', 'sp', 'hvs', 'ivs', 'ps']

    zrows = jnp.zeros((ACC_N, HF), jnp.float32)
    z16 = jnp.zeros((ACC_N, 16), jnp.float32)
    ones_rows = jnp.zeros((K, 16), jnp.float32).at[:, 0].set(1.0)

    dsts9 = jnp.concatenate([lists[k][1] for k in order])
    h16 = _hist(dsts9, ones_rows, z16)
    counts = h16.reshape(NLISTS, ACC_N, 16)[:, :, 0]
    dinv9, cinv9 = _scale(counts)
    dinv = {k: dinv9[i, :N].reshape(N, 1) for i, k in enumerate(order[:3])}
    cinv = {k: cinv9[i + 3, :N].reshape(N, 1)
            for i, k in enumerate(order[3:])}

    def segsum(x, lk):
        src2, dstp = lists[lk]
        y = _segsum(x.reshape(2 * N, HF), src2, dstp, zrows)
        return y.reshape(2, ACC_N, HF)

    def gcn(name, x, lk, relu=True, kw=256):
        w = p[name + '_W']
        if kw == 128:
            w = jnp.pad(w, ((0, 128 - w.shape[0]), (0, 0)))
        mm = _mm128 if kw == 128 else _mm256
        h = mm(x, w, dinv[lk])
        y = segsum(h, lk)
        comb = _gcn_relu if relu else _gcn_lin
        return comb(y, h, dinv[lk], p[name + '_b'].reshape(1, H))

    def sage(name, x_src, x_dst, lk):
        s = segsum(x_src, lk)
        dw = x_dst.shape[1]
        if dw < 128:
            x_dst = _pad_cols(x_dst, 128)
            wr = jnp.pad(p[name + '_Wr'], ((0, 128 - dw), (0, 0)))
            dw = 128
        else:
            wr = p[name + '_Wr']
        fn = _sage128 if dw == 128 else _sage256
        return fn(s, cinv[lk], x_dst, p[name + '_Wl'].reshape(2, HF, H),
                  p[name + '_bl'].reshape(1, H), wr)

    def bn(name, x):
        st = _bnstat(x)
        return _bnapply(x, st, p[name + '_g'].reshape(1, H),
                        p[name + '_b'].reshape(1, H))

    sx = gcn('state_conv1', _pad_cols(state_x, 128), 'ss', kw=128)
    sx = bn('state_norm1', sx)
    gx = sage('state_history_cfg_conv1', sx, game_x, 'hsv')
    gx = sage('state_in_cfg_conv1', sx, gx, 'isv')
    px = sage('state_to_pc_conv1', sx, pc_x, 'sp')
    gx = gcn('cfg_conv1', gx, 'vv')
    gx = bn('cfg_norm1', gx)
    px = gcn('pc_conv1', px, 'pp')
    px = bn('pc_norm1', px)
    sx = sage('cfg_history_state_conv1', gx, sx, 'hvs')
    sx = sage('cfg_in_state_conv1', gx, sx, 'ivs')
    sx = sage('pc_to_state_conv1', px, sx, 'ps')
    sx = gcn('state_conv2', sx, 'ss')
    sx = bn('state_norm2', sx)
    gx = sage('state_history_cfg_conv2', sx, gx, 'hsv')
    gx = sage('state_in_cfg_conv2', sx, gx, 'isv')
    px = sage('state_to_pc_conv2', sx, px, 'sp')
    gx = gcn('cfg_conv2', gx, 'vv')
    gx = bn('cfg_norm2', gx)
    px = gcn('pc_conv2', px, 'pp', relu=False)
    px = bn('pc_norm2', px)
    sx = sage('cfg_history_state_conv2', gx, sx, 'hvs')
    sx = sage('cfg_in_state_conv2', gx, sx, 'ivs')
    sx = sage('pc_to_state_conv2', px, sx, 'ps')
    sx = gcn('state_conv3', sx, 'ss')
    sx = bn('state_norm3', sx)

    w1 = jnp.pad(p['lin_W'], ((0, 0), (0, 128 - 32)))
    b1 = jnp.pad(p['lin_b'], (0, 128 - 32)).reshape(1, 128)
    w2 = jnp.pad(p['lin_last_W'], ((0, 128 - 32), (0, 127)))
    b2 = jnp.pad(p['lin_last_b'], (0, 127)).reshape(1, 128)
    logit = _logits(sx, w1, b1, w2, b2)
    return _lsm(logit)


# speedup vs baseline: 2.8924x; 1.0011x over previous
import functools

import jax
import jax.numpy as jnp
from jax import lax
from jax.experimental import pallas as pl
from jax.experimental.pallas import tpu as pltpu
from jax.experimental.pallas import tpu_sc as plsc

N = 10000
H = 256
E = 160000
HF = 128
K = 128
NT = 16
EPAD = 163840
NCHUNK = EPAD // (NT * K)
ACC_N = 10240
TRASH = 10016
NLISTS = 9
BN_EPS = 1e-5

_mesh = plsc.VectorSubcoreMesh(core_axis_name="c", subcore_axis_name="s")


NBUF = 2
NPHASE = 2
ROWS_T = NCHUNK * K // 128
ROWS_P = ROWS_T // NPHASE


def _segsum_body(x2, src2, dst2, zrows_hbm, out, srcall, dstall,
                 b0, b1, idx_s, idx_d, acc, g0, g1):
    c = lax.axis_index("c")
    s = lax.axis_index("s")
    bufs = (b0, b1)
    gsem = (g0, g1)
    zr = ACC_N // NT
    pltpu.sync_copy(zrows_hbm.at[pl.ds(s * zr, zr)], acc.at[pl.ds(s * zr, zr)])
    pltpu.sync_copy(dst2.at[pl.ds(s * ROWS_T, ROWS_T), :], dstall)
    plsc.subcore_barrier()

    for h in range(NPHASE):
        pltpu.sync_copy(
            src2.at[pl.ds(c * NT * ROWS_T + s * ROWS_T + h * ROWS_P,
                          ROWS_P), :], srcall)

        def group(g, carry):
            for j in range(K // 16):
                idx_s[pl.ds(j * 16, 16)] = srcall[g, pl.ds(j * 16, 16)]
                idx_d[pl.ds(j * 16, 16)] = dstall[h * ROWS_P + g,
                                                  pl.ds(j * 16, 16)]
            pltpu.async_copy(x2.at[idx_s], bufs[0], gsem[0]).wait()
            pltpu.sync_copy(bufs[0], acc.at[idx_d], add=True)
            return carry

        lax.fori_loop(0, ROWS_P, group, 0)
    plsc.subcore_barrier()
    pltpu.sync_copy(acc.at[pl.ds(s * zr, zr)],
                    out.at[pl.ds(c * ACC_N + s * zr, zr), :])


_segsum = functools.partial(
    pl.kernel,
    out_type=jax.ShapeDtypeStruct((2 * ACC_N, HF), jnp.float32),
    mesh=_mesh,
    scratch_types=[
        pltpu.VMEM((ROWS_P, K), jnp.int32),
        pltpu.VMEM((ROWS_T, K), jnp.int32),
        pltpu.VMEM((K, HF), jnp.float32),
        pltpu.VMEM((K, HF), jnp.float32),
        pltpu.VMEM((K,), jnp.int32),
        pltpu.VMEM((K,), jnp.int32),
        pltpu.VMEM_SHARED((ACC_N, HF), jnp.float32),
    ] + [pltpu.SemaphoreType.DMA] * 2,
)(_segsum_body)


_HC = 8192


def _histtc_body(d_ref, o_ref):
    @pl.when(pl.program_id(1) == 0)
    def _():
        o_ref[...] = jnp.zeros_like(o_ref)

    d = d_ref[...]
    io = lax.broadcasted_iota(jnp.int32, (1, 128), 1)
    ohq = (d // 128 == io).astype(jnp.float32)
    ohr = (d % 128 == io).astype(jnp.float32)
    o_ref[0] += lax.dot_general(ohq, ohr, (((0,), (0,)), ((), ())),
                                preferred_element_type=jnp.float32)


_histtc = pl.pallas_call(
    _histtc_body,
    grid=(NLISTS, EPAD // _HC),
    in_specs=[pl.BlockSpec((_HC, 1), lambda l, g: (l * (EPAD // _HC) + g, 0))],
    out_specs=pl.BlockSpec((1, 128, 128), lambda l, g: (l, 0, 0)),
    out_shape=jax.ShapeDtypeStruct((NLISTS, 128, 128), jnp.float32),
    compiler_params=pltpu.CompilerParams(
        dimension_semantics=("arbitrary", "arbitrary")),
)


_BN_ROWS = 2000


def _row_spec(width):
    return pl.BlockSpec((_BN_ROWS, width), lambda i: (i, 0))


def _const_spec(shape):
    return pl.BlockSpec(shape, lambda i: tuple(0 for _ in shape))


def _tc_call(body, in_specs, out_width, grid=N // _BN_ROWS, seq=False):
    return pl.pallas_call(
        body,
        grid=(grid,),
        in_specs=in_specs,
        out_specs=_row_spec(out_width),
        out_shape=jax.ShapeDtypeStruct((N, out_width), jnp.float32),
        compiler_params=pltpu.CompilerParams(
            dimension_semantics=("arbitrary" if seq else "parallel",)),
    )


def _mm_body(x_ref, w_ref, sc_ref, o_ref):
    x = x_ref[...] * sc_ref[...]
    o_ref[...] = jnp.dot(x, w_ref[...], preferred_element_type=jnp.float32)


def _make_mm(kw):
    return _tc_call(_mm_body,
                    [_row_spec(kw), _const_spec((kw, H)), _row_spec(1)], H)


_mm128 = _make_mm(128)
_mm256 = _make_mm(256)


def _sage_body(yc_ref, ci_ref, xd_ref, wl_ref, bl_ref, wr_ref, o_ref):
    ci = ci_ref[...]
    agg = jnp.concatenate([yc_ref[0], yc_ref[1]], axis=-1) * ci
    t = (jnp.dot(agg, wl_ref[...], preferred_element_type=jnp.float32)
         + jnp.dot(xd_ref[...], wr_ref[...], preferred_element_type=jnp.float32)
         + bl_ref[...])
    nrm = jnp.maximum(jnp.sqrt(jnp.sum(t * t, axis=-1, keepdims=True)), 1e-12)
    o_ref[...] = jnp.maximum(t / nrm, 0.0)


def _make_sage(dw):
    return _tc_call(
        _sage_body,
        [pl.BlockSpec((2, _BN_ROWS, HF), lambda i: (0, i, 0)), _row_spec(1),
         _row_spec(dw), _const_spec((H, H)), _const_spec((1, H)),
         _const_spec((dw, H))], H)


_sage128 = _make_sage(128)
_sage256 = _make_sage(256)


def _gcn_body(relu, yc_ref, hp_ref, dv_ref, b_ref, o_ref):
    y = jnp.concatenate([yc_ref[0], yc_ref[1]], axis=-1) + hp_ref[...]
    t = y * dv_ref[...] + b_ref[...]
    o_ref[...] = jnp.maximum(t, 0.0) if relu else t


def _make_gcn(relu):
    return _tc_call(
        functools.partial(_gcn_body, relu),
        [pl.BlockSpec((2, _BN_ROWS, HF), lambda i: (0, i, 0)), _row_spec(H),
         _row_spec(1), _const_spec((1, H))], H)


_gcn_relu = _make_gcn(True)
_gcn_lin = _make_gcn(False)


def _bnstat_body(x_ref, o_ref):
    @pl.when(pl.program_id(0) == 0)
    def _():
        o_ref[...] = jnp.zeros_like(o_ref)
    x = x_ref[...]
    o_ref[0, :] += jnp.sum(x, axis=0)
    o_ref[1, :] += jnp.sum(x * x, axis=0)


_bnstat = pl.pallas_call(
    _bnstat_body,
    grid=(N // _BN_ROWS,),
    in_specs=[_row_spec(H)],
    out_specs=_const_spec((2, H)),
    out_shape=jax.ShapeDtypeStruct((2, H), jnp.float32),
    compiler_params=pltpu.CompilerParams(dimension_semantics=("arbitrary",)),
)


def _bnapply_body(x_ref, st_ref, g_ref, b_ref, o_ref):
    m = st_ref[0, :] / N
    v = st_ref[1, :] / N - m * m
    scale = lax.rsqrt(v + BN_EPS) * g_ref[0, :]
    o_ref[...] = (x_ref[...] - m[None, :]) * scale[None, :] + b_ref[...]


_bnapply = _tc_call(
    _bnapply_body,
    [_row_spec(H), _const_spec((2, H)), _const_spec((1, H)),
     _const_spec((1, H))], H)


def _scale_body(cnt_ref, dinv_ref, cinv_ref):
    cnt = cnt_ref[...]
    dinv_ref[...] = lax.rsqrt(cnt + 1.0)
    cinv_ref[...] = 1.0 / jnp.maximum(cnt, 1.0)


_scale = pl.pallas_call(
    _scale_body,
    in_specs=[pl.BlockSpec((NLISTS, ACC_N), lambda: (0, 0))],
    out_specs=[pl.BlockSpec((NLISTS, ACC_N), lambda: (0, 0))] * 2,
    out_shape=[jax.ShapeDtypeStruct((NLISTS, ACC_N), jnp.float32)] * 2,
)


def _logits_body(x_ref, w1_ref, b1_ref, w2_ref, b2_ref, o_ref):
    t = jnp.dot(x_ref[...], w1_ref[...], preferred_element_type=jnp.float32)
    t = jnp.maximum(t + b1_ref[...], 0.0)
    o_ref[...] = jnp.dot(t, w2_ref[...],
                         preferred_element_type=jnp.float32) + b2_ref[...]


_logits = _tc_call(
    _logits_body,
    [_row_spec(H), _const_spec((H, 128)), _const_spec((1, 128)),
     _const_spec((128, 128)), _const_spec((1, 128))], 128)


def _lsm_body(x_ref, o_ref):
    x = x_ref[:, 0:1]
    m = jnp.max(x)
    lse = m + jnp.log(jnp.sum(jnp.exp(x - m)))
    o_ref[...] = x - lse


_lsm = pl.pallas_call(
    _lsm_body,
    in_specs=[pl.BlockSpec((N, 128), lambda: (0, 0))],
    out_specs=pl.BlockSpec((N, 1), lambda: (0, 0)),
    out_shape=jax.ShapeDtypeStruct((N, 1), jnp.float32),
)


def _pad_cols(x, w):
    return jnp.pad(x, ((0, 0), (0, w - x.shape[1])))


def _prep_edges(ei):
    src = ei[0].astype(jnp.int32)
    dst = ei[1].astype(jnp.int32)
    srcp = jnp.concatenate([src, jnp.zeros((EPAD - E,), jnp.int32)])
    dstp = jnp.concatenate([dst, jnp.full((EPAD - E,), TRASH, jnp.int32)])
    src2 = jnp.concatenate([srcp * 2, srcp * 2 + 1])
    return src2.reshape(2 * NT * ROWS_T, K), dstp.reshape(NT * ROWS_T, K)


def kernel(game_x, state_x, pc_x, edge_index_v_v, edge_type_v_v,
           edge_index_history_v_s, edge_index_history_s_v,
           edge_attr_history_v_s, edge_index_in_v_s, edge_index_in_s_v,
           edge_index_s_s, edge_index_pc_pc, edge_index_pc_s,
           edge_index_s_pc, params):
    p = params
    del edge_type_v_v, edge_attr_history_v_s

    lists = {
        'ss': _prep_edges(edge_index_s_s),
        'vv': _prep_edges(edge_index_v_v),
        'pp': _prep_edges(edge_index_pc_pc),
        'hsv': _prep_edges(edge_index_history_s_v),
        'isv': _prep_edges(edge_index_in_s_v),
        'sp': _prep_edges(edge_index_s_pc),
        'hvs': _prep_edges(edge_index_history_v_s),
        'ivs': _prep_edges(edge_index_in_v_s),
        'ps': _prep_edges(edge_index_pc_s),
    }
    order = ['ss', 'vv', 'pp', 'hsv', 'isv', 'sp', 'hvs', 'ivs', 'ps']

    zrows = jnp.zeros((ACC_N, HF), jnp.float32)

    dsts_col = jnp.concatenate(
        [lists[k][1] for k in order], axis=0).reshape(NLISTS * EPAD, 1)
    h2d = _histtc(dsts_col)
    counts = h2d.reshape(NLISTS, 128 * 128)[:, :ACC_N]
    dinv9, cinv9 = _scale(counts)
    dinv = {k: dinv9[i, :N].reshape(N, 1) for i, k in enumerate(order[:3])}
    cinv = {k: cinv9[i + 3, :N].reshape(N, 1)
            for i, k in enumerate(order[3:])}

    def segsum(x, lk):
        src2, dstp = lists[lk]
        y = _segsum(x.reshape(2 * N, HF), src2, dstp, zrows)
        return y.reshape(2, ACC_N, HF)

    def gcn(name, x, lk, relu=True, kw=256):
        w = p[name + '_W']
        if kw == 128:
            w = jnp.pad(w, ((0, 128 - w.shape[0]), (0, 0)))
        mm = _mm128 if kw == 128 else _mm256
        h = mm(x, w, dinv[lk])
        y = segsum(h, lk)
        comb = _gcn_relu if relu else _gcn_lin
        return comb(y, h, dinv[lk], p[name + '_b'].reshape(1, H))

    def sage(name, x_src, x_dst, lk):
        s = segsum(x_src, lk)
        dw = x_dst.shape[1]
        if dw < 128:
            x_dst = _pad_cols(x_dst, 128)
            wr = jnp.pad(p[name + '_Wr'], ((0, 128 - dw), (0, 0)))
            dw = 128
        else:
            wr = p[name + '_Wr']
        fn = _sage128 if dw == 128 else _sage256
        return fn(s, cinv[lk], x_dst, p[name + '_Wl'],
                  p[name + '_bl'].reshape(1, H), wr)

    def bn(name, x):
        st = _bnstat(x)
        return _bnapply(x, st, p[name + '_g'].reshape(1, H),
                        p[name + '_b'].reshape(1, H))

    sx = gcn('state_conv1', _pad_cols(state_x, 128), 'ss', kw=128)
    sx = bn('state_norm1', sx)
    gx = sage('state_history_cfg_conv1', sx, game_x, 'hsv')
    gx = sage('state_in_cfg_conv1', sx, gx, 'isv')
    px = sage('state_to_pc_conv1', sx, pc_x, 'sp')
    gx = gcn('cfg_conv1', gx, 'vv')
    gx = bn('cfg_norm1', gx)
    px = gcn('pc_conv1', px, 'pp')
    px = bn('pc_norm1', px)
    sx = sage('cfg_history_state_conv1', gx, sx, 'hvs')
    sx = sage('cfg_in_state_conv1', gx, sx, 'ivs')
    sx = sage('pc_to_state_conv1', px, sx, 'ps')
    sx = gcn('state_conv2', sx, 'ss')
    sx = bn('state_norm2', sx)
    gx = sage('state_history_cfg_conv2', sx, gx, 'hsv')
    gx = sage('state_in_cfg_conv2', sx, gx, 'isv')
    px = sage('state_to_pc_conv2', sx, px, 'sp')
    gx = gcn('cfg_conv2', gx, 'vv')
    gx = bn('cfg_norm2', gx)
    px = gcn('pc_conv2', px, 'pp', relu=False)
    px = bn('pc_norm2', px)
    sx = sage('cfg_history_state_conv2', gx, sx, 'hvs')
    sx = sage('cfg_in_state_conv2', gx, sx, 'ivs')
    sx = sage('pc_to_state_conv2', px, sx, 'ps')
    sx = gcn('state_conv3', sx, 'ss')
    sx = bn('state_norm3', sx)

    w1 = jnp.pad(p['lin_W'], ((0, 0), (0, 128 - 32)))
    b1 = jnp.pad(p['lin_b'], (0, 128 - 32)).reshape(1, 128)
    w2 = jnp.pad(p['lin_last_W'], ((0, 128 - 32), (0, 127)))
    b2 = jnp.pad(p['lin_last_b'], (0, 127)).reshape(1, 128)
    logit = _logits(sx, w1, b1, w2, b2)
    return _lsm(logit)


# speedup vs baseline: 3.3885x; 1.1715x over previous
import functools

import jax
import jax.numpy as jnp
from jax import lax
from jax.experimental import pallas as pl
from jax.experimental.pallas import tpu as pltpu
from jax.experimental.pallas import tpu_sc as plsc

N = 10000
H = 256
E = 160000
HF = 128
K = 128
NT = 16
EPAD = 163840
NCHUNK = EPAD // (NT * K)
ACC_N = 10240
TRASH = 10016
NLISTS = 9
BN_EPS = 1e-5

_mesh = plsc.VectorSubcoreMesh(core_axis_name="c", subcore_axis_name="s")


NBUF = 2
NPHASE = 2
ROWS_T = NCHUNK * K // 128
ROWS_P = ROWS_T // NPHASE


def _segsum_body(x2, src2, dst2, zrows_hbm, out, srcall, dstall,
                 b0, b1, i0, i1, idx_d, acc, g0, g1):
    c = lax.axis_index("c")
    s = lax.axis_index("s")
    bufs = (b0, b1)
    idxs = (i0, i1)
    gsem = (g0, g1)
    zr = ACC_N // NT
    pltpu.sync_copy(zrows_hbm.at[pl.ds(s * zr, zr)], acc.at[pl.ds(s * zr, zr)])
    pltpu.sync_copy(dst2.at[pl.ds(s * ROWS_T, ROWS_T), :], dstall)
    plsc.subcore_barrier()

    def cp_idx(dst_ref, src_ref, row):
        for j in range(K // 16):
            dst_ref[pl.ds(j * 16, 16)] = src_ref[row, pl.ds(j * 16, 16)]

    for h in range(NPHASE):
        pltpu.sync_copy(
            src2.at[pl.ds(c * NT * ROWS_T + s * ROWS_T + h * ROWS_P,
                          ROWS_P), :], srcall)
        for b in range(NBUF):
            cp_idx(idxs[b], srcall, b)
            pltpu.make_async_copy(x2.at[idxs[b]], bufs[b], gsem[b]).start()

        def group(g2i, carry):
            for b in range(NBUF):
                g = g2i * NBUF + b
                pltpu.make_async_copy(x2.at[idxs[b]], bufs[b],
                                      gsem[b]).wait()
                cp_idx(idx_d, dstall, h * ROWS_P + g)
                pltpu.sync_copy(bufs[b], acc.at[idx_d], add=True)

                @pl.when(g + NBUF < ROWS_P)
                def _():
                    cp_idx(idxs[b], srcall, g + NBUF)
                    pltpu.make_async_copy(x2.at[idxs[b]], bufs[b],
                                          gsem[b]).start()
            return carry

        lax.fori_loop(0, ROWS_P // NBUF, group, 0)
    plsc.subcore_barrier()
    pltpu.sync_copy(acc.at[pl.ds(s * zr, zr)],
                    out.at[pl.ds(c * ACC_N + s * zr, zr), :])


_segsum = functools.partial(
    pl.kernel,
    out_type=jax.ShapeDtypeStruct((2 * ACC_N, HF), jnp.float32),
    mesh=_mesh,
    scratch_types=[
        pltpu.VMEM((ROWS_P, K), jnp.int32),
        pltpu.VMEM((ROWS_T, K), jnp.int32),
        pltpu.VMEM((K, HF), jnp.float32),
        pltpu.VMEM((K, HF), jnp.float32),
        pltpu.VMEM((K,), jnp.int32),
        pltpu.VMEM((K,), jnp.int32),
        pltpu.VMEM((K,), jnp.int32),
        pltpu.VMEM_SHARED((ACC_N, HF), jnp.float32),
    ] + [pltpu.SemaphoreType.DMA] * 2,
)(_segsum_body)


_HC = 8192


def _histtc_body(d_ref, o_ref):
    @pl.when(pl.program_id(1) == 0)
    def _():
        o_ref[...] = jnp.zeros_like(o_ref)

    d = d_ref[...]
    io = lax.broadcasted_iota(jnp.int32, (1, 128), 1)
    ohq = (d // 128 == io).astype(jnp.float32)
    ohr = (d % 128 == io).astype(jnp.float32)
    o_ref[0] += lax.dot_general(ohq, ohr, (((0,), (0,)), ((), ())),
                                preferred_element_type=jnp.float32)


_histtc = pl.pallas_call(
    _histtc_body,
    grid=(NLISTS, EPAD // _HC),
    in_specs=[pl.BlockSpec((_HC, 1), lambda l, g: (l * (EPAD // _HC) + g, 0))],
    out_specs=pl.BlockSpec((1, 128, 128), lambda l, g: (l, 0, 0)),
    out_shape=jax.ShapeDtypeStruct((NLISTS, 128, 128), jnp.float32),
    compiler_params=pltpu.CompilerParams(
        dimension_semantics=("arbitrary", "arbitrary")),
)


_BN_ROWS = 2000


def _row_spec(width):
    return pl.BlockSpec((_BN_ROWS, width), lambda i: (i, 0))


def _const_spec(shape):
    return pl.BlockSpec(shape, lambda i: tuple(0 for _ in shape))


def _tc_call(body, in_specs, out_width, grid=N // _BN_ROWS, seq=False):
    return pl.pallas_call(
        body,
        grid=(grid,),
        in_specs=in_specs,
        out_specs=_row_spec(out_width),
        out_shape=jax.ShapeDtypeStruct((N, out_width), jnp.float32),
        compiler_params=pltpu.CompilerParams(
            dimension_semantics=("arbitrary" if seq else "parallel",)),
    )


def _mm_body(x_ref, w_ref, sc_ref, o_ref):
    x = x_ref[...] * sc_ref[...]
    o_ref[...] = jnp.dot(x, w_ref[...], preferred_element_type=jnp.float32)


def _make_mm(kw):
    return _tc_call(_mm_body,
                    [_row_spec(kw), _const_spec((kw, H)), _row_spec(1)], H)


_mm128 = _make_mm(128)
_mm256 = _make_mm(256)


def _sage_body(yc_ref, ci_ref, xd_ref, wl_ref, bl_ref, wr_ref, o_ref):
    ci = ci_ref[...]
    agg = jnp.concatenate([yc_ref[0], yc_ref[1]], axis=-1) * ci
    t = (jnp.dot(agg, wl_ref[...], preferred_element_type=jnp.float32)
         + jnp.dot(xd_ref[...], wr_ref[...], preferred_element_type=jnp.float32)
         + bl_ref[...])
    nrm = jnp.maximum(jnp.sqrt(jnp.sum(t * t, axis=-1, keepdims=True)), 1e-12)
    o_ref[...] = jnp.maximum(t / nrm, 0.0)


def _make_sage(dw):
    return _tc_call(
        _sage_body,
        [pl.BlockSpec((2, _BN_ROWS, HF), lambda i: (0, i, 0)), _row_spec(1),
         _row_spec(dw), _const_spec((H, H)), _const_spec((1, H)),
         _const_spec((dw, H))], H)


_sage128 = _make_sage(128)
_sage256 = _make_sage(256)


def _gcn_body(relu, yc_ref, hp_ref, dv_ref, b_ref, o_ref):
    y = jnp.concatenate([yc_ref[0], yc_ref[1]], axis=-1) + hp_ref[...]
    t = y * dv_ref[...] + b_ref[...]
    o_ref[...] = jnp.maximum(t, 0.0) if relu else t


def _make_gcn(relu):
    return _tc_call(
        functools.partial(_gcn_body, relu),
        [pl.BlockSpec((2, _BN_ROWS, HF), lambda i: (0, i, 0)), _row_spec(H),
         _row_spec(1), _const_spec((1, H))], H)


_gcn_relu = _make_gcn(True)
_gcn_lin = _make_gcn(False)


def _bnstat_body(x_ref, o_ref):
    @pl.when(pl.program_id(0) == 0)
    def _():
        o_ref[...] = jnp.zeros_like(o_ref)
    x = x_ref[...]
    o_ref[0, :] += jnp.sum(x, axis=0)
    o_ref[1, :] += jnp.sum(x * x, axis=0)


_bnstat = pl.pallas_call(
    _bnstat_body,
    grid=(N // _BN_ROWS,),
    in_specs=[_row_spec(H)],
    out_specs=_const_spec((2, H)),
    out_shape=jax.ShapeDtypeStruct((2, H), jnp.float32),
    compiler_params=pltpu.CompilerParams(dimension_semantics=("arbitrary",)),
)


def _bnapply_body(x_ref, st_ref, g_ref, b_ref, o_ref):
    m = st_ref[0, :] / N
    v = st_ref[1, :] / N - m * m
    scale = lax.rsqrt(v + BN_EPS) * g_ref[0, :]
    o_ref[...] = (x_ref[...] - m[None, :]) * scale[None, :] + b_ref[...]


_bnapply = _tc_call(
    _bnapply_body,
    [_row_spec(H), _const_spec((2, H)), _const_spec((1, H)),
     _const_spec((1, H))], H)


def _scale_body(cnt_ref, dinv_ref, cinv_ref):
    cnt = cnt_ref[...]
    dinv_ref[...] = lax.rsqrt(cnt + 1.0)
    cinv_ref[...] = 1.0 / jnp.maximum(cnt, 1.0)


_scale = pl.pallas_call(
    _scale_body,
    in_specs=[pl.BlockSpec((NLISTS, ACC_N), lambda: (0, 0))],
    out_specs=[pl.BlockSpec((NLISTS, ACC_N), lambda: (0, 0))] * 2,
    out_shape=[jax.ShapeDtypeStruct((NLISTS, ACC_N), jnp.float32)] * 2,
)


def _logits_body(x_ref, w1_ref, b1_ref, w2_ref, b2_ref, o_ref):
    t = jnp.dot(x_ref[...], w1_ref[...], preferred_element_type=jnp.float32)
    t = jnp.maximum(t + b1_ref[...], 0.0)
    o_ref[...] = jnp.dot(t, w2_ref[...],
                         preferred_element_type=jnp.float32) + b2_ref[...]


_logits = _tc_call(
    _logits_body,
    [_row_spec(H), _const_spec((H, 128)), _const_spec((1, 128)),
     _const_spec((128, 128)), _const_spec((1, 128))], 128)


def _lsm_body(x_ref, o_ref):
    x = x_ref[:, 0:1]
    m = jnp.max(x)
    lse = m + jnp.log(jnp.sum(jnp.exp(x - m)))
    o_ref[...] = x - lse


_lsm = pl.pallas_call(
    _lsm_body,
    in_specs=[pl.BlockSpec((N, 128), lambda: (0, 0))],
    out_specs=pl.BlockSpec((N, 1), lambda: (0, 0)),
    out_shape=jax.ShapeDtypeStruct((N, 1), jnp.float32),
)


def _pad_cols(x, w):
    return jnp.pad(x, ((0, 0), (0, w - x.shape[1])))


def _prep_edges(ei):
    src = ei[0].astype(jnp.int32)
    dst = ei[1].astype(jnp.int32)
    srcp = jnp.concatenate([src, jnp.zeros((EPAD - E,), jnp.int32)])
    dstp = jnp.concatenate([dst, jnp.full((EPAD - E,), TRASH, jnp.int32)])
    src2 = jnp.concatenate([srcp * 2, srcp * 2 + 1])
    return src2.reshape(2 * NT * ROWS_T, K), dstp.reshape(NT * ROWS_T, K)


def kernel(game_x, state_x, pc_x, edge_index_v_v, edge_type_v_v,
           edge_index_history_v_s, edge_index_history_s_v,
           edge_attr_history_v_s, edge_index_in_v_s, edge_index_in_s_v,
           edge_index_s_s, edge_index_pc_pc, edge_index_pc_s,
           edge_index_s_pc, params):
    p = params
    del edge_type_v_v, edge_attr_history_v_s

    lists = {
        'ss': _prep_edges(edge_index_s_s),
        'vv': _prep_edges(edge_index_v_v),
        'pp': _prep_edges(edge_index_pc_pc),
        'hsv': _prep_edges(edge_index_history_s_v),
        'isv': _prep_edges(edge_index_in_s_v),
        'sp': _prep_edges(edge_index_s_pc),
        'hvs': _prep_edges(edge_index_history_v_s),
        'ivs': _prep_edges(edge_index_in_v_s),
        'ps': _prep_edges(edge_index_pc_s),
    }
    order = ['ss', 'vv', 'pp', 'hsv', 'isv', 'sp', 'hvs', 'ivs', 'ps']

    zrows = jnp.zeros((ACC_N, HF), jnp.float32)

    dsts_col = jnp.concatenate(
        [lists[k][1] for k in order], axis=0).reshape(NLISTS * EPAD, 1)
    h2d = _histtc(dsts_col)
    counts = h2d.reshape(NLISTS, 128 * 128)[:, :ACC_N]
    dinv9, cinv9 = _scale(counts)
    dinv = {k: dinv9[i, :N].reshape(N, 1) for i, k in enumerate(order[:3])}
    cinv = {k: cinv9[i + 3, :N].reshape(N, 1)
            for i, k in enumerate(order[3:])}

    def segsum(x, lk):
        src2, dstp = lists[lk]
        y = _segsum(x.reshape(2 * N, HF), src2, dstp, zrows)
        return y.reshape(2, ACC_N, HF)

    def gcn(name, x, lk, relu=True, kw=256):
        w = p[name + '_W']
        if kw == 128:
            w = jnp.pad(w, ((0, 128 - w.shape[0]), (0, 0)))
        mm = _mm128 if kw == 128 else _mm256
        h = mm(x, w, dinv[lk])
        y = segsum(h, lk)
        comb = _gcn_relu if relu else _gcn_lin
        return comb(y, h, dinv[lk], p[name + '_b'].reshape(1, H))

    def sage(name, x_src, x_dst, lk):
        s = segsum(x_src, lk)
        dw = x_dst.shape[1]
        if dw < 128:
            x_dst = _pad_cols(x_dst, 128)
            wr = jnp.pad(p[name + '_Wr'], ((0, 128 - dw), (0, 0)))
            dw = 128
        else:
            wr = p[name + '_Wr']
        fn = _sage128 if dw == 128 else _sage256
        return fn(s, cinv[lk], x_dst, p[name + '_Wl'],
                  p[name + '_bl'].reshape(1, H), wr)

    def bn(name, x):
        st = _bnstat(x)
        return _bnapply(x, st, p[name + '_g'].reshape(1, H),
                        p[name + '_b'].reshape(1, H))

    sx = gcn('state_conv1', _pad_cols(state_x, 128), 'ss', kw=128)
    sx = bn('state_norm1', sx)
    gx = sage('state_history_cfg_conv1', sx, game_x, 'hsv')
    gx = sage('state_in_cfg_conv1', sx, gx, 'isv')
    px = sage('state_to_pc_conv1', sx, pc_x, 'sp')
    gx = gcn('cfg_conv1', gx, 'vv')
    gx = bn('cfg_norm1', gx)
    px = gcn('pc_conv1', px, 'pp')
    px = bn('pc_norm1', px)
    sx = sage('cfg_history_state_conv1', gx, sx, 'hvs')
    sx = sage('cfg_in_state_conv1', gx, sx, 'ivs')
    sx = sage('pc_to_state_conv1', px, sx, 'ps')
    sx = gcn('state_conv2', sx, 'ss')
    sx = bn('state_norm2', sx)
    gx = sage('state_history_cfg_conv2', sx, gx, 'hsv')
    gx = sage('state_in_cfg_conv2', sx, gx, 'isv')
    px = sage('state_to_pc_conv2', sx, px, 'sp')
    gx = gcn('cfg_conv2', gx, 'vv')
    gx = bn('cfg_norm2', gx)
    px = gcn('pc_conv2', px, 'pp', relu=False)
    px = bn('pc_norm2', px)
    sx = sage('cfg_history_state_conv2', gx, sx, 'hvs')
    sx = sage('cfg_in_state_conv2', gx, sx, 'ivs')
    sx = sage('pc_to_state_conv2', px, sx, 'ps')
    sx = gcn('state_conv3', sx, 'ss')
    sx = bn('state_norm3', sx)

    w1 = jnp.pad(p['lin_W'], ((0, 0), (0, 128 - 32)))
    b1 = jnp.pad(p['lin_b'], (0, 128 - 32)).reshape(1, 128)
    w2 = jnp.pad(p['lin_last_W'], ((0, 128 - 32), (0, 127)))
    b2 = jnp.pad(p['lin_last_b'], (0, 127)).reshape(1, 128)
    logit = _logits(sx, w1, b1, w2, b2)
    return _lsm(logit)
